# Initial kernel scaffold; baseline (speedup 1.0000x reference)
import jax
import jax.numpy as jnp
from jax.experimental import pallas as pl


def kernel(x_taxon, x_sotu, ei_taxon_taxon, ei_taxon_sotu, W1l, b1, W1r, W2l, b2, W2r, W3l, b3, W3r, Wlin, blin):
    raise NotImplementedError("write your pallas kernel here")


# speedup vs baseline: 6.8411x; 6.8411x over previous
import jax
import jax.numpy as jnp
from jax import lax
from jax.experimental import pallas as pl
from jax.experimental.pallas import tpu as pltpu
from jax.experimental.pallas import tpu_sc as plsc

_N = 10000
_NPAD = 10240
_D = 128
_NSUB = 16
_CHUNK = 80
_SLAB = 25
_ROWS_PER_TILE = _NPAD // _NSUB
_NFULL = _ROWS_PER_TILE // _CHUNK

_mesh = plsc.VectorSubcoreMesh(core_axis_name="c", subcore_axis_name="s")


def _make_sc_seg(total_edges, count):
    half = total_edges // 2
    per_w = half // _NSUB
    nslabs = per_w // (_SLAB * _CHUNK)
    assert nslabs * _SLAB * _CHUNK == per_w

    out_type = [jax.ShapeDtypeStruct((2 * _NPAD, _D), jnp.float32)]
    if count:
        out_type.append(jax.ShapeDtypeStruct((2 * _NPAD,), jnp.float32))

    scratch = [
        pltpu.VMEM_SHARED((_NPAD, _D), jnp.float32),
        pltpu.VMEM((_SLAB, _CHUNK), jnp.int32),
        pltpu.VMEM((_SLAB, _CHUNK), jnp.int32),
        pltpu.VMEM((_CHUNK, _D), jnp.float32),
        pltpu.SemaphoreType.DMA,
    ]
    if count:
        scratch.insert(1, pltpu.VMEM_SHARED((_NPAD,), jnp.float32))
        scratch.insert(5, pltpu.VMEM((_ROWS_PER_TILE,), jnp.float32))
        scratch.insert(6, pltpu.VMEM((_CHUNK,), jnp.float32))

    def body(*refs):
        if count:
            (x_hbm, src_hbm, dst_hbm, s_out, c_out,
             acc, cnt, src_t, dst_t, rows_v, cstage, ones_v, sem) = refs
        else:
            (x_hbm, src_hbm, dst_hbm, s_out,
             acc, src_t, dst_t, rows_v, sem) = refs

        cid = lax.axis_index("c")
        sid = lax.axis_index("s")
        wid = cid * _NSUB + sid
        zero16 = jnp.zeros((16,), jnp.float32)

        def _zrows(i, carry):
            for j in range(_D // 16):
                rows_v[i, pl.ds(j * 16, 16)] = zero16
            return carry
        lax.fori_loop(0, _CHUNK, _zrows, 0)

        row0 = sid * _ROWS_PER_TILE

        def _zacc(j, carry):
            r = pl.multiple_of(row0 + j * _CHUNK, 8)
            pltpu.sync_copy(rows_v, acc.at[pl.ds(r, _CHUNK)])
            return carry
        lax.fori_loop(0, _NFULL, _zacc, 0)

        if count:
            def _zc(i, carry):
                cstage[pl.ds(i * 16, 16)] = zero16
                return carry
            lax.fori_loop(0, _ROWS_PER_TILE // 16, _zc, 0)
            pltpu.sync_copy(cstage, cnt.at[pl.ds(row0, _ROWS_PER_TILE)])
            one16 = jnp.ones((16,), jnp.float32)
            for i in range(_CHUNK // 16):
                ones_v[pl.ds(i * 16, 16)] = one16

        plsc.subcore_barrier()

        @pl.loop(0, nslabs)
        def _slab(o):
            w3 = wid * nslabs + o
            pltpu.sync_copy(src_hbm.at[w3], src_t)
            pltpu.sync_copy(dst_hbm.at[w3], dst_t)

            @pl.loop(0, _SLAB)
            def _step(j):
                pltpu.async_copy(x_hbm.at[src_t.at[j]], rows_v, sem).wait()
                pltpu.sync_copy(rows_v, acc.at[dst_t.at[j]], add=True)
                if count:
                    pltpu.sync_copy(ones_v, cnt.at[dst_t.at[j]], add=True)

        plsc.subcore_barrier()

        obase = cid * _NPAD + row0

        def _wout(j, carry):
            ra = pl.multiple_of(row0 + j * _CHUNK, 8)
            ro = pl.multiple_of(obase + j * _CHUNK, 8)
            pltpu.sync_copy(acc.at[pl.ds(ra, _CHUNK)], rows_v)
            pltpu.sync_copy(rows_v, s_out.at[pl.ds(ro, _CHUNK)])
            return carry
        lax.fori_loop(0, _NFULL, _wout, 0)
        if count:
            pltpu.sync_copy(cnt.at[pl.ds(row0, _ROWS_PER_TILE)], cstage)
            pltpu.sync_copy(
                cstage,
                c_out.at[pl.ds(pl.multiple_of(obase, 8), _ROWS_PER_TILE)])

    return pl.kernel(body, mesh=_mesh, out_type=tuple(out_type),
                     scratch_types=tuple(scratch))


_seg_both = _make_sc_seg(2 * 320000, count=True)
_seg_half = _make_sc_seg(320000, count=False)


def _tile_idx(a):
    return a.astype(jnp.int32).reshape(-1, _SLAB, _CHUNK)


_R = 1000


def _tc_layer1(s1, c1, x, wl_t, wr_t, b):
    def body(s_ref, c_ref, x_ref, wl_ref, wr_ref, b_ref, o_ref):
        cnt = jnp.maximum(c_ref[...], 1.0)
        mean = s_ref[...] / cnt
        acc = jnp.dot(mean, wl_ref[...], preferred_element_type=jnp.float32)
        acc += jnp.dot(x_ref[...], wr_ref[...],
                       preferred_element_type=jnp.float32)
        o_ref[...] = jnp.maximum(acc + b_ref[...], 0.0)

    return pl.pallas_call(
        body,
        grid=(_N // _R,),
        in_specs=[
            pl.BlockSpec((_R, _D), lambda i: (i, 0)),
            pl.BlockSpec((_R, 1), lambda i: (i, 0)),
            pl.BlockSpec((_R, _D), lambda i: (i, 0)),
            pl.BlockSpec((_D, _D), lambda i: (0, 0)),
            pl.BlockSpec((_D, _D), lambda i: (0, 0)),
            pl.BlockSpec((1, _D), lambda i: (0, 0)),
        ],
        out_specs=pl.BlockSpec((_R, _D), lambda i: (i, 0)),
        out_shape=jax.ShapeDtypeStruct((_N, _D), jnp.float32),
    )(s1, c1, x, wl_t, wr_t, b)


def _tc_final(s2, c2, x_sotu, s3a, s3b,
              w2l_t, w2r_t, b2, w3l_t, w3r_t, b3, wlin_t, blin):
    def body(s2_ref, c_ref, x_ref, a_ref, p_ref,
             w2l_ref, w2r_ref, b2_ref, w3l_ref, w3r_ref, b3_ref,
             wlin_ref, blin_ref, o_ref):
        cnt = jnp.maximum(c_ref[...], 1.0)
        mean2 = s2_ref[...] / cnt
        sotu = jnp.dot(mean2, w2l_ref[...], preferred_element_type=jnp.float32)
        sotu += jnp.dot(x_ref[...], w2r_ref[...],
                        preferred_element_type=jnp.float32)
        sotu = jnp.maximum(sotu + b2_ref[...], 0.0)
        mean3 = (a_ref[...] + p_ref[...]) / cnt
        h = jnp.dot(mean3, w3l_ref[...], preferred_element_type=jnp.float32)
        h += jnp.dot(sotu, w3r_ref[...], preferred_element_type=jnp.float32)
        h = jnp.maximum(h + b3_ref[...], 0.0)
        o_ref[...] = jnp.dot(h, wlin_ref[...],
                             preferred_element_type=jnp.float32) + blin_ref[...]

    row = lambda i: (i, 0)
    fixed = lambda i: (0, 0)
    return pl.pallas_call(
        body,
        grid=(_N // _R,),
        in_specs=[
            pl.BlockSpec((_R, _D), row),
            pl.BlockSpec((_R, 1), row),
            pl.BlockSpec((_R, _D), row),
            pl.BlockSpec((_R, _D), row),
            pl.BlockSpec((_R, _D), row),
            pl.BlockSpec((_D, _D), fixed),
            pl.BlockSpec((_D, _D), fixed),
            pl.BlockSpec((1, _D), fixed),
            pl.BlockSpec((_D, _D), fixed),
            pl.BlockSpec((_D, _D), fixed),
            pl.BlockSpec((1, _D), fixed),
            pl.BlockSpec((_D, _D), fixed),
            pl.BlockSpec((1, _D), fixed),
        ],
        out_specs=pl.BlockSpec((_R, _D), row),
        out_shape=jax.ShapeDtypeStruct((_N, _D), jnp.float32),
    )(s2, c2, x_sotu, s3a, s3b,
      w2l_t, w2r_t, b2, w3l_t, w3r_t, b3, wlin_t, blin)


def kernel(x_taxon, x_sotu, ei_taxon_taxon, ei_taxon_sotu,
           W1l, b1, W1r, W2l, b2, W2r, W3l, b3, W3r, Wlin, blin):
    src_a = _tile_idx(jnp.concatenate([ei_taxon_taxon[0], ei_taxon_sotu[0]]))
    dst_a = _tile_idx(jnp.concatenate([ei_taxon_taxon[1], ei_taxon_sotu[1]]))
    S, C = _seg_both(x_taxon, src_a, dst_a)

    c_tt = C[:_N].reshape(_N, 1)
    c_ts = C[_NPAD:_NPAD + _N].reshape(_N, 1)

    taxon_x = _tc_layer1(S[:_N], c_tt, x_taxon,
                         W1l.T, W1r.T, b1.reshape(1, _D))

    (S3,) = _seg_half(taxon_x, _tile_idx(ei_taxon_sotu[0]),
                      _tile_idx(ei_taxon_sotu[1]))

    return _tc_final(S[_NPAD:_NPAD + _N], c_ts, x_sotu,
                     S3[:_N], S3[_NPAD:_NPAD + _N],
                     W2l.T, W2r.T, b2.reshape(1, _D),
                     W3l.T, W3r.T, b3.reshape(1, _D),
                     Wlin.T, blin.reshape(1, _D))


# speedup vs baseline: 8.6605x; 1.2660x over previous
import jax
import jax.numpy as jnp
from jax import lax
from jax.experimental import pallas as pl
from jax.experimental.pallas import tpu as pltpu
from jax.experimental.pallas import tpu_sc as plsc

_N = 10000
_NPAD = 10240
_D = 128
_NSUB = 16
_SLAB = 10
_WCHUNK = 80
_ROWS_PER_TILE = _NPAD // _NSUB
_NFULL = _ROWS_PER_TILE // _WCHUNK

_mesh = plsc.VectorSubcoreMesh(core_axis_name="c", subcore_axis_name="s")


def _make_sc_seg(total_edges, chunk, count):
    half = total_edges // 2
    per_w = half // _NSUB
    nslabs = per_w // (_SLAB * chunk)
    assert nslabs * _SLAB * chunk == per_w and _SLAB % 2 == 0

    out_type = [jax.ShapeDtypeStruct((2 * _NPAD, _D), jnp.float32)]
    if count:
        out_type.append(jax.ShapeDtypeStruct((2 * _NPAD,), jnp.float32))

    scratch = [
        pltpu.VMEM_SHARED((_NPAD, _D), jnp.float32),
        pltpu.VMEM((_SLAB, chunk), jnp.int32),
        pltpu.VMEM((_SLAB, chunk), jnp.int32),
        pltpu.VMEM((chunk, _D), jnp.float32),
        pltpu.VMEM((chunk, _D), jnp.float32),
        pltpu.VMEM((_WCHUNK, _D), jnp.float32),
        pltpu.SemaphoreType.DMA,
        pltpu.SemaphoreType.DMA,
    ]
    if count:
        scratch.insert(1, pltpu.VMEM_SHARED((_NPAD,), jnp.float32))
        scratch.insert(7, pltpu.VMEM((_ROWS_PER_TILE,), jnp.float32))
        scratch.insert(8, pltpu.VMEM((chunk,), jnp.float32))

    def body(*refs):
        if count:
            (x_hbm, src_hbm, dst_hbm, s_out, c_out,
             acc, cnt, src_t, dst_t, buf_a, buf_b, stage_v,
             cstage, ones_v, sem_a, sem_b) = refs
        else:
            (x_hbm, src_hbm, dst_hbm, s_out,
             acc, src_t, dst_t, buf_a, buf_b, stage_v,
             sem_a, sem_b) = refs

        cid = lax.axis_index("c")
        sid = lax.axis_index("s")
        wid = cid * _NSUB + sid
        zero16 = jnp.zeros((16,), jnp.float32)

        def _zrows(i, carry):
            for j in range(_D // 16):
                stage_v[i, pl.ds(j * 16, 16)] = zero16
            return carry
        lax.fori_loop(0, _WCHUNK, _zrows, 0)

        row0 = sid * _ROWS_PER_TILE

        def _zacc(j, carry):
            r = pl.multiple_of(row0 + j * _WCHUNK, 8)
            pltpu.sync_copy(stage_v, acc.at[pl.ds(r, _WCHUNK)])
            return carry
        lax.fori_loop(0, _NFULL, _zacc, 0)

        if count:
            def _zc(i, carry):
                cstage[pl.ds(i * 16, 16)] = zero16
                return carry
            lax.fori_loop(0, _ROWS_PER_TILE // 16, _zc, 0)
            pltpu.sync_copy(cstage, cnt.at[pl.ds(row0, _ROWS_PER_TILE)])
            one16 = jnp.ones((16,), jnp.float32)
            for i in range(chunk // 16):
                ones_v[pl.ds(i * 16, 16)] = one16

        plsc.subcore_barrier()

        def _scat(buf, j):
            pltpu.sync_copy(buf, acc.at[dst_t.at[j]], add=True)
            if count:
                pltpu.sync_copy(ones_v, cnt.at[dst_t.at[j]], add=True)

        @pl.loop(0, nslabs)
        def _slab(o):
            w3 = wid * nslabs + o
            pltpu.sync_copy(src_hbm.at[w3], src_t)
            pltpu.sync_copy(dst_hbm.at[w3], dst_t)

            pltpu.async_copy(x_hbm.at[src_t.at[0]], buf_a, sem_a)

            @pl.loop(0, _SLAB // 2)
            def _pair(p):
                j0 = p * 2
                pltpu.async_copy(x_hbm.at[src_t.at[j0 + 1]], buf_b, sem_b)
                pltpu.make_async_copy(
                    x_hbm.at[src_t.at[j0]], buf_a, sem_a).wait()
                _scat(buf_a, j0)

                @pl.when(p < _SLAB // 2 - 1)
                def _refill():
                    pltpu.async_copy(
                        x_hbm.at[src_t.at[j0 + 2]], buf_a, sem_a)

                pltpu.make_async_copy(
                    x_hbm.at[src_t.at[j0 + 1]], buf_b, sem_b).wait()
                _scat(buf_b, j0 + 1)

        plsc.subcore_barrier()

        obase = cid * _NPAD + row0

        def _wout(j, carry):
            ra = pl.multiple_of(row0 + j * _WCHUNK, 8)
            ro = pl.multiple_of(obase + j * _WCHUNK, 8)
            pltpu.sync_copy(acc.at[pl.ds(ra, _WCHUNK)], stage_v)
            pltpu.sync_copy(stage_v, s_out.at[pl.ds(ro, _WCHUNK)])
            return carry
        lax.fori_loop(0, _NFULL, _wout, 0)
        if count:
            pltpu.sync_copy(cnt.at[pl.ds(row0, _ROWS_PER_TILE)], cstage)
            pltpu.sync_copy(
                cstage,
                c_out.at[pl.ds(pl.multiple_of(obase, 8), _ROWS_PER_TILE)])

    return pl.kernel(body, mesh=_mesh, out_type=tuple(out_type),
                     scratch_types=tuple(scratch))


_CHUNK_A = 80
_CHUNK_B = 40
_seg_both = _make_sc_seg(2 * 320000, _CHUNK_A, count=True)
_seg_half = _make_sc_seg(320000, _CHUNK_B, count=False)


def _tile_idx(a, chunk):
    return a.astype(jnp.int32).reshape(-1, _SLAB, chunk)


_R = 1000


def _tc_layer1(s1, c1, x, wl_t, wr_t, b):
    def body(s_ref, c_ref, x_ref, wl_ref, wr_ref, b_ref, o_ref):
        cnt = jnp.maximum(c_ref[...], 1.0)
        mean = s_ref[...] / cnt
        acc = jnp.dot(mean, wl_ref[...], preferred_element_type=jnp.float32)
        acc += jnp.dot(x_ref[...], wr_ref[...],
                       preferred_element_type=jnp.float32)
        o_ref[...] = jnp.maximum(acc + b_ref[...], 0.0)

    return pl.pallas_call(
        body,
        grid=(_N // _R,),
        in_specs=[
            pl.BlockSpec((_R, _D), lambda i: (i, 0)),
            pl.BlockSpec((_R, 1), lambda i: (i, 0)),
            pl.BlockSpec((_R, _D), lambda i: (i, 0)),
            pl.BlockSpec((_D, _D), lambda i: (0, 0)),
            pl.BlockSpec((_D, _D), lambda i: (0, 0)),
            pl.BlockSpec((1, _D), lambda i: (0, 0)),
        ],
        out_specs=pl.BlockSpec((_R, _D), lambda i: (i, 0)),
        out_shape=jax.ShapeDtypeStruct((_N, _D), jnp.float32),
    )(s1, c1, x, wl_t, wr_t, b)


def _tc_final(s2, c2, x_sotu, s3a, s3b,
              w2l_t, w2r_t, b2, w3l_t, w3r_t, b3, wlin_t, blin):
    def body(s2_ref, c_ref, x_ref, a_ref, p_ref,
             w2l_ref, w2r_ref, b2_ref, w3l_ref, w3r_ref, b3_ref,
             wlin_ref, blin_ref, o_ref):
        cnt = jnp.maximum(c_ref[...], 1.0)
        mean2 = s2_ref[...] / cnt
        sotu = jnp.dot(mean2, w2l_ref[...], preferred_element_type=jnp.float32)
        sotu += jnp.dot(x_ref[...], w2r_ref[...],
                        preferred_element_type=jnp.float32)
        sotu = jnp.maximum(sotu + b2_ref[...], 0.0)
        mean3 = (a_ref[...] + p_ref[...]) / cnt
        h = jnp.dot(mean3, w3l_ref[...], preferred_element_type=jnp.float32)
        h += jnp.dot(sotu, w3r_ref[...], preferred_element_type=jnp.float32)
        h = jnp.maximum(h + b3_ref[...], 0.0)
        o_ref[...] = jnp.dot(h, wlin_ref[...],
                             preferred_element_type=jnp.float32) + blin_ref[...]

    row = lambda i: (i, 0)
    fixed = lambda i: (0, 0)
    return pl.pallas_call(
        body,
        grid=(_N // _R,),
        in_specs=[
            pl.BlockSpec((_R, _D), row),
            pl.BlockSpec((_R, 1), row),
            pl.BlockSpec((_R, _D), row),
            pl.BlockSpec((_R, _D), row),
            pl.BlockSpec((_R, _D), row),
            pl.BlockSpec((_D, _D), fixed),
            pl.BlockSpec((_D, _D), fixed),
            pl.BlockSpec((1, _D), fixed),
            pl.BlockSpec((_D, _D), fixed),
            pl.BlockSpec((_D, _D), fixed),
            pl.BlockSpec((1, _D), fixed),
            pl.BlockSpec((_D, _D), fixed),
            pl.BlockSpec((1, _D), fixed),
        ],
        out_specs=pl.BlockSpec((_R, _D), row),
        out_shape=jax.ShapeDtypeStruct((_N, _D), jnp.float32),
    )(s2, c2, x_sotu, s3a, s3b,
      w2l_t, w2r_t, b2, w3l_t, w3r_t, b3, wlin_t, blin)


def kernel(x_taxon, x_sotu, ei_taxon_taxon, ei_taxon_sotu,
           W1l, b1, W1r, W2l, b2, W2r, W3l, b3, W3r, Wlin, blin):
    src_a = _tile_idx(jnp.concatenate([ei_taxon_taxon[0], ei_taxon_sotu[0]]),
                      _CHUNK_A)
    dst_a = _tile_idx(jnp.concatenate([ei_taxon_taxon[1], ei_taxon_sotu[1]]),
                      _CHUNK_A)
    S, C = _seg_both(x_taxon, src_a, dst_a)

    c_tt = C[:_N].reshape(_N, 1)
    c_ts = C[_NPAD:_NPAD + _N].reshape(_N, 1)

    taxon_x = _tc_layer1(S[:_N], c_tt, x_taxon,
                         W1l.T, W1r.T, b1.reshape(1, _D))

    (S3,) = _seg_half(taxon_x, _tile_idx(ei_taxon_sotu[0], _CHUNK_B),
                      _tile_idx(ei_taxon_sotu[1], _CHUNK_B))

    return _tc_final(S[_NPAD:_NPAD + _N], c_ts, x_sotu,
                     S3[:_N], S3[_NPAD:_NPAD + _N],
                     W2l.T, W2r.T, b2.reshape(1, _D),
                     W3l.T, W3r.T, b3.reshape(1, _D),
                     Wlin.T, blin.reshape(1, _D))


# speedup vs baseline: 10.5063x; 1.2131x over previous
import jax
import jax.numpy as jnp
from jax import lax
from jax.experimental import pallas as pl
from jax.experimental.pallas import tpu as pltpu
from jax.experimental.pallas import tpu_sc as plsc

_N = 10000
_NPAD = 10240
_D = 128
_NSUB = 16
_SLAB = 25
_WCHUNK = 80
_ROWS_PER_TILE = _NPAD // _NSUB
_NFULL = _ROWS_PER_TILE // _WCHUNK

_mesh = plsc.VectorSubcoreMesh(core_axis_name="c", subcore_axis_name="s")


def _make_sc_seg(total_edges, chunk, count):
    half = total_edges // 2
    per_w = half // _NSUB
    nslabs = per_w // (_SLAB * chunk)
    assert nslabs * _SLAB * chunk == per_w and _SLAB % 2 == 1

    out_type = [jax.ShapeDtypeStruct((2 * _NPAD, _D), jnp.float32)]
    if count:
        out_type.append(jax.ShapeDtypeStruct((2 * _NPAD,), jnp.float32))

    scratch = [
        pltpu.VMEM_SHARED((_NPAD, _D), jnp.float32),
        pltpu.VMEM((_SLAB, chunk), jnp.int32),
        pltpu.VMEM((_SLAB, chunk), jnp.int32),
        pltpu.VMEM((chunk, _D), jnp.float32),
        pltpu.VMEM((chunk, _D), jnp.float32),
        pltpu.SemaphoreType.DMA,
        pltpu.SemaphoreType.DMA,
    ]
    if count:
        scratch.insert(1, pltpu.VMEM_SHARED((_NPAD,), jnp.float32))
        scratch.insert(6, pltpu.VMEM((_ROWS_PER_TILE,), jnp.float32))
        scratch.insert(7, pltpu.VMEM((chunk,), jnp.float32))

    assert chunk == _WCHUNK

    def body(*refs):
        if count:
            (x_hbm, src_hbm, dst_hbm, s_out, c_out,
             acc, cnt, src_t, dst_t, buf_a, buf_b,
             cstage, ones_v, sem_a, sem_b) = refs
        else:
            (x_hbm, src_hbm, dst_hbm, s_out,
             acc, src_t, dst_t, buf_a, buf_b,
             sem_a, sem_b) = refs
        stage_v = buf_a

        cid = lax.axis_index("c")
        sid = lax.axis_index("s")
        wid = cid * _NSUB + sid
        zero16 = jnp.zeros((16,), jnp.float32)

        def _zrows(i, carry):
            for j in range(_D // 16):
                stage_v[i, pl.ds(j * 16, 16)] = zero16
            return carry
        lax.fori_loop(0, _WCHUNK, _zrows, 0)

        row0 = sid * _ROWS_PER_TILE

        def _zacc(j, carry):
            r = pl.multiple_of(row0 + j * _WCHUNK, 8)
            pltpu.sync_copy(stage_v, acc.at[pl.ds(r, _WCHUNK)])
            return carry
        lax.fori_loop(0, _NFULL, _zacc, 0)

        if count:
            def _zc(i, carry):
                cstage[pl.ds(i * 16, 16)] = zero16
                return carry
            lax.fori_loop(0, _ROWS_PER_TILE // 16, _zc, 0)
            pltpu.sync_copy(cstage, cnt.at[pl.ds(row0, _ROWS_PER_TILE)])
            one16 = jnp.ones((16,), jnp.float32)
            for i in range(chunk // 16):
                ones_v[pl.ds(i * 16, 16)] = one16

        plsc.subcore_barrier()

        def _scat(buf, j):
            pltpu.sync_copy(buf, acc.at[dst_t.at[j]], add=True)
            if count:
                pltpu.sync_copy(ones_v, cnt.at[dst_t.at[j]], add=True)

        @pl.loop(0, nslabs)
        def _slab(o):
            w3 = wid * nslabs + o
            pltpu.sync_copy(src_hbm.at[w3], src_t)
            pltpu.sync_copy(dst_hbm.at[w3], dst_t)

            pltpu.async_copy(x_hbm.at[src_t.at[0]], buf_a, sem_a)

            @pl.loop(0, (_SLAB - 1) // 2)
            def _pair(p):
                j0 = p * 2
                pltpu.async_copy(x_hbm.at[src_t.at[j0 + 1]], buf_b, sem_b)
                pltpu.make_async_copy(
                    x_hbm.at[src_t.at[j0]], buf_a, sem_a).wait()
                _scat(buf_a, j0)
                pltpu.async_copy(x_hbm.at[src_t.at[j0 + 2]], buf_a, sem_a)
                pltpu.make_async_copy(
                    x_hbm.at[src_t.at[j0 + 1]], buf_b, sem_b).wait()
                _scat(buf_b, j0 + 1)

            pltpu.make_async_copy(
                x_hbm.at[src_t.at[_SLAB - 1]], buf_a, sem_a).wait()
            _scat(buf_a, _SLAB - 1)

        plsc.subcore_barrier()

        obase = cid * _NPAD + row0

        def _wout(j, carry):
            ra = pl.multiple_of(row0 + j * _WCHUNK, 8)
            ro = pl.multiple_of(obase + j * _WCHUNK, 8)
            pltpu.sync_copy(acc.at[pl.ds(ra, _WCHUNK)], stage_v)
            pltpu.sync_copy(stage_v, s_out.at[pl.ds(ro, _WCHUNK)])
            return carry
        lax.fori_loop(0, _NFULL, _wout, 0)
        if count:
            pltpu.sync_copy(cnt.at[pl.ds(row0, _ROWS_PER_TILE)], cstage)
            pltpu.sync_copy(
                cstage,
                c_out.at[pl.ds(pl.multiple_of(obase, 8), _ROWS_PER_TILE)])

    return pl.kernel(body, mesh=_mesh, out_type=tuple(out_type),
                     scratch_types=tuple(scratch))


_CHUNK_A = 80
_CHUNK_B = 80
_seg_both = _make_sc_seg(2 * 320000, _CHUNK_A, count=True)
_seg_half = _make_sc_seg(320000, _CHUNK_B, count=False)


def _tile_idx(a, chunk):
    return a.astype(jnp.int32).reshape(-1, _SLAB, chunk)


_R = 1000


def _tc_layer1(s1, c1, x, wl_t, wr_t, b):
    def body(s_ref, c_ref, x_ref, wl_ref, wr_ref, b_ref, o_ref):
        cnt = jnp.maximum(c_ref[...], 1.0)
        mean = s_ref[...] / cnt
        acc = jnp.dot(mean, wl_ref[...], preferred_element_type=jnp.float32)
        acc += jnp.dot(x_ref[...], wr_ref[...],
                       preferred_element_type=jnp.float32)
        o_ref[...] = jnp.maximum(acc + b_ref[...], 0.0)

    return pl.pallas_call(
        body,
        grid=(_N // _R,),
        in_specs=[
            pl.BlockSpec((_R, _D), lambda i: (i, 0)),
            pl.BlockSpec((_R, 1), lambda i: (i, 0)),
            pl.BlockSpec((_R, _D), lambda i: (i, 0)),
            pl.BlockSpec((_D, _D), lambda i: (0, 0)),
            pl.BlockSpec((_D, _D), lambda i: (0, 0)),
            pl.BlockSpec((1, _D), lambda i: (0, 0)),
        ],
        out_specs=pl.BlockSpec((_R, _D), lambda i: (i, 0)),
        out_shape=jax.ShapeDtypeStruct((_N, _D), jnp.float32),
    )(s1, c1, x, wl_t, wr_t, b)


def _tc_final(s2, c2, x_sotu, s3a, s3b,
              w2l_t, w2r_t, b2, w3l_t, w3r_t, b3, wlin_t, blin):
    def body(s2_ref, c_ref, x_ref, a_ref, p_ref,
             w2l_ref, w2r_ref, b2_ref, w3l_ref, w3r_ref, b3_ref,
             wlin_ref, blin_ref, o_ref):
        cnt = jnp.maximum(c_ref[...], 1.0)
        mean2 = s2_ref[...] / cnt
        sotu = jnp.dot(mean2, w2l_ref[...], preferred_element_type=jnp.float32)
        sotu += jnp.dot(x_ref[...], w2r_ref[...],
                        preferred_element_type=jnp.float32)
        sotu = jnp.maximum(sotu + b2_ref[...], 0.0)
        mean3 = (a_ref[...] + p_ref[...]) / cnt
        h = jnp.dot(mean3, w3l_ref[...], preferred_element_type=jnp.float32)
        h += jnp.dot(sotu, w3r_ref[...], preferred_element_type=jnp.float32)
        h = jnp.maximum(h + b3_ref[...], 0.0)
        o_ref[...] = jnp.dot(h, wlin_ref[...],
                             preferred_element_type=jnp.float32) + blin_ref[...]

    row = lambda i: (i, 0)
    fixed = lambda i: (0, 0)
    return pl.pallas_call(
        body,
        grid=(_N // _R,),
        in_specs=[
            pl.BlockSpec((_R, _D), row),
            pl.BlockSpec((_R, 1), row),
            pl.BlockSpec((_R, _D), row),
            pl.BlockSpec((_R, _D), row),
            pl.BlockSpec((_R, _D), row),
            pl.BlockSpec((_D, _D), fixed),
            pl.BlockSpec((_D, _D), fixed),
            pl.BlockSpec((1, _D), fixed),
            pl.BlockSpec((_D, _D), fixed),
            pl.BlockSpec((_D, _D), fixed),
            pl.BlockSpec((1, _D), fixed),
            pl.BlockSpec((_D, _D), fixed),
            pl.BlockSpec((1, _D), fixed),
        ],
        out_specs=pl.BlockSpec((_R, _D), row),
        out_shape=jax.ShapeDtypeStruct((_N, _D), jnp.float32),
    )(s2, c2, x_sotu, s3a, s3b,
      w2l_t, w2r_t, b2, w3l_t, w3r_t, b3, wlin_t, blin)


def kernel(x_taxon, x_sotu, ei_taxon_taxon, ei_taxon_sotu,
           W1l, b1, W1r, W2l, b2, W2r, W3l, b3, W3r, Wlin, blin):
    src_a = _tile_idx(jnp.concatenate([ei_taxon_taxon[0], ei_taxon_sotu[0]]),
                      _CHUNK_A)
    dst_a = _tile_idx(jnp.concatenate([ei_taxon_taxon[1], ei_taxon_sotu[1]]),
                      _CHUNK_A)
    S, C = _seg_both(x_taxon, src_a, dst_a)

    c_tt = C[:_N].reshape(_N, 1)
    c_ts = C[_NPAD:_NPAD + _N].reshape(_N, 1)

    taxon_x = _tc_layer1(S[:_N], c_tt, x_taxon,
                         W1l.T, W1r.T, b1.reshape(1, _D))

    (S3,) = _seg_half(taxon_x, _tile_idx(ei_taxon_sotu[0], _CHUNK_B),
                      _tile_idx(ei_taxon_sotu[1], _CHUNK_B))

    return _tc_final(S[_NPAD:_NPAD + _N], c_ts, x_sotu,
                     S3[:_N], S3[_NPAD:_NPAD + _N],
                     W2l.T, W2r.T, b2.reshape(1, _D),
                     W3l.T, W3r.T, b3.reshape(1, _D),
                     Wlin.T, blin.reshape(1, _D))
